# baseline (speedup 1.0000x reference)
import functools

import jax
import jax.numpy as jnp
from jax import lax
from jax.experimental import pallas as pl
from jax.experimental.pallas import tpu as pltpu
from jax.experimental.pallas import tpu_sc as plsc

VOCAB = 128
MM_HIDDEN = 2048
TXT_HIDDEN = 2048
EPS = 1e-6

NC, NS = 2, 16
NW = NC * NS
TOKENS = 4 * 2048
SC_TOKENS = 512
TC_TOKENS = TOKENS - SC_TOKENS
TC_BLK = 2560
MERGE_BLK = 512
CHUNK = 16
NBUF = 2


KBLK = 512
NKB = MM_HIDDEN // KBLK


def _lut_body(table_ref, w_ref, proj_ref, out_ref, acc_ref):
    k = pl.program_id(0)
    sl = pl.ds(k * KBLK, KBLK)
    x = table_ref[...]
    var = jnp.mean(x * x, axis=-1, keepdims=True)
    scale = lax.rsqrt(var + EPS)
    xk = table_ref[:, sl] * scale * (1.0 + w_ref[0, sl])
    part = lax.dot_general(
        xk, proj_ref[...],
        dimension_numbers=(((1,), (1,)), ((), ())),
        preferred_element_type=jnp.float32,
    )

    @pl.when(k == 0)
    def _init():
        acc_ref[...] = part

    @pl.when(k > 0)
    def _accum():
        acc_ref[...] += part

    @pl.when(k == NKB - 1)
    def _finish():
        y = acc_ref[...]
        var2 = jnp.mean(y * y, axis=-1, keepdims=True)
        out_ref[...] = y * lax.rsqrt(var2 + EPS)


def _compute_lut(embedding_table, hard_norm_weight, proj_weight):
    return pl.pallas_call(
        _lut_body,
        grid=(NKB,),
        in_specs=[
            pl.BlockSpec((VOCAB, MM_HIDDEN), lambda k: (0, 0)),
            pl.BlockSpec((1, MM_HIDDEN), lambda k: (0, 0)),
            pl.BlockSpec((TXT_HIDDEN, KBLK), lambda k: (0, k)),
        ],
        out_specs=pl.BlockSpec((VOCAB, TXT_HIDDEN), lambda k: (0, 0)),
        out_shape=jax.ShapeDtypeStruct((VOCAB, TXT_HIDDEN), jnp.float32),
        scratch_shapes=[pltpu.VMEM((VOCAB, TXT_HIDDEN), jnp.float32)],
    )(embedding_table, hard_norm_weight.reshape(1, MM_HIDDEN), proj_weight)


def _tc_gather_body(ids_ref, lut_ref, out_ref):
    ids_col = ids_ref[0].reshape(TC_BLK, 1)
    vocab_iota = lax.broadcasted_iota(jnp.int32, (TC_BLK, VOCAB), 1)
    onehot = (ids_col == vocab_iota).astype(jnp.float32)
    out_ref[...] = lax.dot_general(
        onehot, lut_ref[...],
        dimension_numbers=(((1,), (0,)), ((), ())),
        preferred_element_type=jnp.float32,
    )


def _tc_gather_full(lut, ids):
    nblk = TC_TOKENS // TC_BLK
    return pl.pallas_call(
        _tc_gather_body,
        grid=(nblk,),
        in_specs=[
            pl.BlockSpec((1, 1, TC_BLK), lambda i: (i, 0, 0)),
            pl.BlockSpec((VOCAB, TXT_HIDDEN), lambda i: (0, 0)),
        ],
        out_specs=pl.BlockSpec((TC_BLK, TXT_HIDDEN), lambda i: (i, 0)),
        out_shape=jax.ShapeDtypeStruct((TOKENS, TXT_HIDDEN), jnp.float32),
    )(ids.reshape(nblk, 1, TC_BLK), lut)


def _merge_body(tc_ref, sc_ref, out_ref):
    del tc_ref
    out_ref[...] = sc_ref[...]


def _merge(tc_full, sc_out):
    nblk = SC_TOKENS // MERGE_BLK
    tc_blocks = TC_TOKENS // MERGE_BLK
    return pl.pallas_call(
        _merge_body,
        grid=(nblk,),
        in_specs=[
            pl.BlockSpec(memory_space=pl.ANY),
            pl.BlockSpec((MERGE_BLK, TXT_HIDDEN), lambda i: (i, 0)),
        ],
        out_specs=pl.BlockSpec((MERGE_BLK, TXT_HIDDEN),
                               lambda i: (tc_blocks + i, 0)),
        out_shape=jax.ShapeDtypeStruct((TOKENS, TXT_HIDDEN), jnp.float32),
        input_output_aliases={0: 0},
    )(tc_full, sc_out)


def _sc_gather_body(nchunk, b_per_w, lut_hbm, ids_hbm, out_hbm,
                    idx_v, rows_v, gsem, wsem):
    wid = lax.axis_index("s") * NC + lax.axis_index("c")
    base = wid * b_per_w
    pltpu.sync_copy(ids_hbm.at[wid], idx_v)

    def gather(c):
        return pltpu.async_copy(
            lut_hbm.at[idx_v.at[c]], rows_v.at[c % NBUF], gsem)

    gh = [None] * nchunk
    wh = [None] * nchunk
    gh[0] = gather(0)
    for c in range(nchunk):
        gh[c].wait()
        wh[c] = pltpu.async_copy(
            rows_v.at[c % NBUF], out_hbm.at[pl.ds(base + c * CHUNK, CHUNK)],
            wsem)
        if c + 1 < nchunk:
            if c + 1 >= NBUF:
                wh[c + 1 - NBUF].wait()
            gh[c + 1] = gather(c + 1)
    for c in range(max(0, nchunk - NBUF), nchunk):
        wh[c].wait()


@functools.lru_cache(maxsize=2)
def _build_sc_gather(sc_tokens):
    b_per_w = sc_tokens // NW
    nchunk = b_per_w // CHUNK
    return pl.kernel(
        functools.partial(_sc_gather_body, nchunk, b_per_w),
        out_type=jax.ShapeDtypeStruct((sc_tokens, TXT_HIDDEN), jnp.float32),
        mesh=plsc.VectorSubcoreMesh(core_axis_name="c", subcore_axis_name="s"),
        scratch_types=[
            pltpu.VMEM((nchunk, CHUNK), jnp.int32),
            pltpu.VMEM((NBUF, CHUNK, TXT_HIDDEN), jnp.float32),
            pltpu.SemaphoreType.DMA,
            pltpu.SemaphoreType.DMA,
        ],
    )


def kernel(input_ids, embedding_table, hard_norm_weight, proj_weight):
    lut = _compute_lut(embedding_table, hard_norm_weight, proj_weight)
    flat = input_ids.reshape(-1)
    sc_ids = flat[TC_TOKENS:].reshape(NW, SC_TOKENS // NW // CHUNK, CHUNK)
    sc_out = _build_sc_gather(SC_TOKENS)(lut, sc_ids)
    tc_full = _tc_gather_full(lut, flat[:TC_TOKENS])
    out = _merge(tc_full, sc_out)
    return out.reshape(input_ids.shape[0], input_ids.shape[1], TXT_HIDDEN)


# speedup vs baseline: 1.0176x; 1.0176x over previous
import functools

import jax
import jax.numpy as jnp
from jax import lax
from jax.experimental import pallas as pl
from jax.experimental.pallas import tpu as pltpu
from jax.experimental.pallas import tpu_sc as plsc

VOCAB = 128
MM_HIDDEN = 2048
TXT_HIDDEN = 2048
EPS = 1e-6

NC, NS = 2, 16
NW = NC * NS
TOKENS = 4 * 2048
SC_TOKENS = 512
TC_TOKENS = TOKENS - SC_TOKENS
TC_BLK = 1920
MERGE_BLK = 512
CHUNK = 16
NBUF = 2


KBLK = 512
NKB = MM_HIDDEN // KBLK


def _lut_body(table_ref, w_ref, proj_ref, out_ref, acc_ref):
    k = pl.program_id(0)
    sl = pl.ds(k * KBLK, KBLK)
    x = table_ref[...]
    var = jnp.mean(x * x, axis=-1, keepdims=True)
    scale = lax.rsqrt(var + EPS)
    xk = table_ref[:, sl] * scale * (1.0 + w_ref[0, sl])
    part = lax.dot_general(
        xk, proj_ref[...],
        dimension_numbers=(((1,), (1,)), ((), ())),
        preferred_element_type=jnp.float32,
    )

    @pl.when(k == 0)
    def _init():
        acc_ref[...] = part

    @pl.when(k > 0)
    def _accum():
        acc_ref[...] += part

    @pl.when(k == NKB - 1)
    def _finish():
        y = acc_ref[...]
        var2 = jnp.mean(y * y, axis=-1, keepdims=True)
        out_ref[...] = y * lax.rsqrt(var2 + EPS)


def _compute_lut(embedding_table, hard_norm_weight, proj_weight):
    return pl.pallas_call(
        _lut_body,
        grid=(NKB,),
        in_specs=[
            pl.BlockSpec((VOCAB, MM_HIDDEN), lambda k: (0, 0)),
            pl.BlockSpec((1, MM_HIDDEN), lambda k: (0, 0)),
            pl.BlockSpec((TXT_HIDDEN, KBLK), lambda k: (0, k)),
        ],
        out_specs=pl.BlockSpec((VOCAB, TXT_HIDDEN), lambda k: (0, 0)),
        out_shape=jax.ShapeDtypeStruct((VOCAB, TXT_HIDDEN), jnp.float32),
        scratch_shapes=[pltpu.VMEM((VOCAB, TXT_HIDDEN), jnp.float32)],
    )(embedding_table, hard_norm_weight.reshape(1, MM_HIDDEN), proj_weight)


def _tc_gather_body(ids_ref, lut_ref, out_ref):
    ids_col = ids_ref[0].reshape(TC_BLK, 1)
    vocab_iota = lax.broadcasted_iota(jnp.int32, (TC_BLK, VOCAB), 1)
    onehot = (ids_col == vocab_iota).astype(jnp.float32)
    out_ref[...] = lax.dot_general(
        onehot, lut_ref[...],
        dimension_numbers=(((1,), (0,)), ((), ())),
        preferred_element_type=jnp.float32,
    )


def _tc_gather_full(lut, ids):
    nblk = TC_TOKENS // TC_BLK
    return pl.pallas_call(
        _tc_gather_body,
        grid=(nblk,),
        in_specs=[
            pl.BlockSpec((1, 1, TC_BLK), lambda i: (i, 0, 0)),
            pl.BlockSpec((VOCAB, TXT_HIDDEN), lambda i: (0, 0)),
        ],
        out_specs=pl.BlockSpec((TC_BLK, TXT_HIDDEN), lambda i: (i, 0)),
        out_shape=jax.ShapeDtypeStruct((TOKENS, TXT_HIDDEN), jnp.float32),
    )(ids.reshape(nblk, 1, TC_BLK), lut)


def _merge_body(tc_ref, sc_ref, out_ref):
    del tc_ref
    out_ref[...] = sc_ref[...]


def _merge(tc_full, sc_out):
    nblk = SC_TOKENS // MERGE_BLK
    tc_blocks = TC_TOKENS // MERGE_BLK
    return pl.pallas_call(
        _merge_body,
        grid=(nblk,),
        in_specs=[
            pl.BlockSpec(memory_space=pl.ANY),
            pl.BlockSpec((MERGE_BLK, TXT_HIDDEN), lambda i: (i, 0)),
        ],
        out_specs=pl.BlockSpec((MERGE_BLK, TXT_HIDDEN),
                               lambda i: (tc_blocks + i, 0)),
        out_shape=jax.ShapeDtypeStruct((TOKENS, TXT_HIDDEN), jnp.float32),
        input_output_aliases={0: 0},
    )(tc_full, sc_out)


def _sc_gather_body(nchunk, b_per_w, lut_hbm, ids_hbm, out_hbm,
                    idx_v, rows_v, gsem, wsem):
    wid = lax.axis_index("s") * NC + lax.axis_index("c")
    base = wid * b_per_w
    pltpu.sync_copy(ids_hbm.at[wid], idx_v)

    def gather(c):
        return pltpu.async_copy(
            lut_hbm.at[idx_v.at[c]], rows_v.at[c % NBUF], gsem)

    gh = [None] * nchunk
    wh = [None] * nchunk
    gh[0] = gather(0)
    for c in range(nchunk):
        gh[c].wait()
        wh[c] = pltpu.async_copy(
            rows_v.at[c % NBUF], out_hbm.at[pl.ds(base + c * CHUNK, CHUNK)],
            wsem)
        if c + 1 < nchunk:
            if c + 1 >= NBUF:
                wh[c + 1 - NBUF].wait()
            gh[c + 1] = gather(c + 1)
    for c in range(max(0, nchunk - NBUF), nchunk):
        wh[c].wait()


@functools.lru_cache(maxsize=2)
def _build_sc_gather(sc_tokens):
    b_per_w = sc_tokens // NW
    nchunk = b_per_w // CHUNK
    return pl.kernel(
        functools.partial(_sc_gather_body, nchunk, b_per_w),
        out_type=jax.ShapeDtypeStruct((sc_tokens, TXT_HIDDEN), jnp.float32),
        mesh=plsc.VectorSubcoreMesh(core_axis_name="c", subcore_axis_name="s"),
        scratch_types=[
            pltpu.VMEM((nchunk, CHUNK), jnp.int32),
            pltpu.VMEM((NBUF, CHUNK, TXT_HIDDEN), jnp.float32),
            pltpu.SemaphoreType.DMA,
            pltpu.SemaphoreType.DMA,
        ],
    )


def kernel(input_ids, embedding_table, hard_norm_weight, proj_weight):
    lut = _compute_lut(embedding_table, hard_norm_weight, proj_weight)
    flat = input_ids.reshape(-1)
    sc_ids = flat[TC_TOKENS:].reshape(NW, SC_TOKENS // NW // CHUNK, CHUNK)
    sc_out = _build_sc_gather(SC_TOKENS)(lut, sc_ids)
    tc_full = _tc_gather_full(lut, flat[:TC_TOKENS])
    out = _merge(tc_full, sc_out)
    return out.reshape(input_ids.shape[0], input_ids.shape[1], TXT_HIDDEN)
